# baseline (speedup 1.0000x reference)
import functools

import jax
import jax.numpy as jnp
from jax import lax
from jax.experimental import pallas as pl
from jax.experimental.pallas import tpu as pltpu
from jax.experimental.pallas import tpu_sc as plsc

V = 32000
D = 768
SS = 16
E = 4
DFF = 4 * D
L = 2048

T_CHUNK = 128
M_BLK = 256
F_BLK = 512
N_BLK = 3200
G_PAD = L + E * M_BLK
NBLK_MAX = G_PAD // M_BLK


def _sc_gather(table, idx, n_out):
    info = plsc.get_sparse_core_info()
    nw = info.num_cores * info.num_subcores
    b_per_w = n_out // nw
    mesh = plsc.VectorSubcoreMesh(core_axis_name="c", subcore_axis_name="s")

    @functools.partial(
        pl.kernel,
        mesh=mesh,
        out_type=jax.ShapeDtypeStruct((n_out, D), jnp.float32),
        scratch_types=[
            pltpu.VMEM((b_per_w,), jnp.int32),
            pltpu.VMEM((b_per_w, D), jnp.float32),
            pltpu.SemaphoreType.DMA,
        ],
    )
    def k(table_hbm, idx_hbm, out_hbm, idx_v, rows_v, sem):
        wid = lax.axis_index("s") * info.num_cores + lax.axis_index("c")
        base = wid * b_per_w
        pltpu.sync_copy(idx_hbm.at[pl.ds(base, b_per_w)], idx_v)
        pltpu.async_copy(table_hbm.at[idx_v], rows_v, sem).wait()
        pltpu.sync_copy(rows_v, out_hbm.at[pl.ds(base, b_per_w)])

    return k(table, idx)


def _sc_route_gather(ti, y):
    info = plsc.get_sparse_core_info()
    nw = info.num_cores * info.num_subcores
    rows_w = L // nw
    mesh = plsc.VectorSubcoreMesh(core_axis_name="c", subcore_axis_name="s")

    @functools.partial(
        pl.kernel,
        mesh=mesh,
        out_type=[
            jax.ShapeDtypeStruct((G_PAD, D), jnp.float32),
            jax.ShapeDtypeStruct((L,), jnp.int32),
            jax.ShapeDtypeStruct((16,), jnp.int32),
        ],
        scratch_types=[
            pltpu.VMEM((L,), jnp.int32),
            pltpu.VMEM((L,), jnp.int32),
            pltpu.VMEM((16,), jnp.int32),
            pltpu.VMEM((rows_w,), jnp.int32),
            pltpu.VMEM((rows_w, D), jnp.float32),
            pltpu.SemaphoreType.DMA,
        ],
    )
    def k(ti_hbm, y_hbm, xg_hbm, inv_hbm, be_hbm,
          ti_v, inv_v, be_v, idx_v, rows_v, sem):
        wid = lax.axis_index("s") * info.num_cores + lax.axis_index("c")
        pltpu.sync_copy(ti_hbm, ti_v)
        i16 = lax.iota(jnp.int32, 16)
        zv = jnp.zeros((16,), jnp.int32)
        one = jnp.ones((16,), jnp.int32)

        def cnt_body(i, accs):
            eid = ti_v[pl.ds(i * 16, 16)]
            return tuple(accs[e] + jnp.where(eid == e, one, zv)
                         for e in range(E))
        accs = lax.fori_loop(0, L // 16, cnt_body, (zv,) * E)

        def lane_sum(vec):
            s = vec[0]
            for k in range(1, 16):
                s = s + vec[k]
            return s

        cnts = [lane_sum(accs[e]) for e in range(E)]

        nb = [lax.shift_right_logical(c + (M_BLK - 1), 8) for c in cnts]
        cb1 = nb[0]
        cb2 = nb[0] + nb[1]
        cb3 = cb2 + nb[2]
        be_v[...] = (jnp.where(i16 >= cb1, one, zv)
                     + jnp.where(i16 >= cb2, one, zv)
                     + jnp.where(i16 >= cb3, one, zv))

        def sc_body(i, bases):
            b0, b1, b2, b3 = bases
            eid = ti_v[pl.ds(i * 16, 16)]
            eks = [eid[k] for k in range(16)]
            rank = zv
            for k in range(16):
                hit = jnp.logical_and(eid == eks[k], i16 > k)
                rank = rank + jnp.where(hit, one, zv)
            base_vec = jnp.where(eid == 0, b0,
                                 jnp.where(eid == 1, b1,
                                           jnp.where(eid == 2, b2, b3)))
            inv_v[pl.ds(i * 16, 16)] = base_vec + rank
            for k in range(16):
                b0 = b0 + (eks[k] == 0).astype(jnp.int32)
                b1 = b1 + (eks[k] == 1).astype(jnp.int32)
                b2 = b2 + (eks[k] == 2).astype(jnp.int32)
                b3 = b3 + (eks[k] == 3).astype(jnp.int32)
            return (b0, b1, b2, b3)
        lax.fori_loop(0, L // 16, sc_body,
                      (jnp.zeros((), jnp.int32), cb1 * M_BLK, cb2 * M_BLK,
                       cb3 * M_BLK))

        base = wid * rows_w
        def cp(j, c):
            idx_v[pl.ds(j * 16, 16)] = inv_v[pl.ds(base + j * 16, 16)]
            return c
        lax.fori_loop(0, rows_w // 16, cp, 0)
        pltpu.sync_copy(y_hbm.at[pl.ds(base, rows_w)], rows_v)
        pltpu.async_copy(rows_v, xg_hbm.at[idx_v], sem).wait()

        @pl.when(wid == 0)
        def _():
            pltpu.sync_copy(inv_v, inv_hbm)
            pltpu.sync_copy(be_v, be_hbm)

    return k(ti, y)


def _ln(o, g, b):
    mu = jnp.mean(o, axis=1, keepdims=True)
    oc = o - mu
    var = jnp.mean(oc * oc, axis=1, keepdims=True)
    return oc * lax.rsqrt(var + 1e-5) * g + b


def _scan_body(fuse_ln, *refs):
    if fuse_ln:
        (yp_ref, moe_ref, twp_ref, g_ref, b_ref, dw_ref, db_ref, w2t_ref,
         bcb_ref, at_ref, dp_ref, rw_ref, rb_ref,
         y_ref, tw_ref, ti_ref, dstate, h_s, x_s) = refs
    else:
        (x_ref, dw_ref, db_ref, w2t_ref, bcb_ref, at_ref,
         dp_ref, rw_ref, rb_ref, y_ref, tw_ref, ti_ref,
         dstate, h_s, x_s) = refs
    gi = pl.program_id(0)

    @pl.when(gi == 0)
    def _():
        h_s[...] = jnp.zeros_like(h_s)

    if fuse_ln:
        xb = _ln(yp_ref[...] + twp_ref[...] * moe_ref[...],
                 g_ref[...], b_ref[...])
        x_s[...] = xb
        x_ref = x_s
    else:
        xb = x_ref[...]
    delta = jnp.dot(xb, dw_ref[...], preferred_element_type=jnp.float32)
    delta = delta + db_ref[...]
    delta = jnp.log(1.0 + jnp.exp(-jnp.abs(delta))) + jnp.maximum(delta, 0.0)
    dstate[...] = delta
    bct = lax.dot_general(w2t_ref[...], xb, (((1,), (1,)), ((), ())),
                          preferred_element_type=jnp.float32)
    bct = bct + bcb_ref[...]

    at = at_ref[...]
    lane_iota = lax.broadcasted_iota(jnp.int32, (1, T_CHUNK), 1)
    SG = 8

    def group(g, h):
        base = g * SG
        d_g = dstate[pl.ds(base, SG), :]
        x_g = x_ref[pl.ds(base, SG), :]
        ys = []
        for k in range(SG):
            t = base + k
            oh = (lane_iota == t).astype(jnp.float32)
            bc_col = jnp.sum(bct * oh, axis=1, keepdims=True)
            b_col = bc_col[0:SS, :]
            c_col = bc_col[SS:2 * SS, :]
            d_t = d_g[k:k + 1, :]
            x_t = x_g[k:k + 1, :]
            a = jnp.exp(jnp.minimum(d_t * at, 2.0))
            bb = jnp.clip(d_t * b_col, -2.0, 2.0)
            h = a * h + bb * x_t
            h = jnp.clip(h, -100.0, 100.0)
            ys.append(jnp.sum(h * c_col, axis=0, keepdims=True))
        y_ref[pl.ds(base, SG), :] = jnp.concatenate(ys, axis=0)
        return h

    h = lax.fori_loop(0, T_CHUNK // SG, group, h_s[...])
    h_s[...] = h
    yb = y_ref[...] + xb * dp_ref[...]
    y_ref[...] = yb

    logits = jnp.dot(yb, rw_ref[...], preferred_element_type=jnp.float32)
    logits = (logits + rb_ref[...])[:, 0:E]
    mx = jnp.max(logits, axis=1, keepdims=True)
    ex = jnp.exp(logits - mx)
    sm = ex / jnp.sum(ex, axis=1, keepdims=True)
    tw = jnp.max(sm, axis=1, keepdims=True)
    iot = lax.broadcasted_iota(jnp.int32, (T_CHUNK, E), 1)
    ti = jnp.min(jnp.where(sm >= tw, iot, E), axis=1, keepdims=True)
    tw_ref[...] = tw
    ti_ref[...] = ti


def _ssm_scan(xargs, dw, db, w2t, bcb, at, dp, rw_p, rb_p, fuse_ln):
    grid = (L // T_CHUNK,)
    blk = pl.BlockSpec((T_CHUNK, D), lambda i: (i, 0))
    col = pl.BlockSpec((T_CHUNK, 1), lambda i: (i, 0))
    row = pl.BlockSpec((1, D), lambda i: (0, 0))
    if fuse_ln:
        xspecs = [blk, blk, col, row, row]
    else:
        xspecs = [blk]
    return pl.pallas_call(
        functools.partial(_scan_body, fuse_ln),
        grid=grid,
        in_specs=xspecs + [
            pl.BlockSpec((D, D), lambda i: (0, 0)),
            row,
            pl.BlockSpec((2 * SS, D), lambda i: (0, 0)),
            pl.BlockSpec((2 * SS, 1), lambda i: (0, 0)),
            pl.BlockSpec((SS, D), lambda i: (0, 0)),
            row,
            pl.BlockSpec((D, 128), lambda i: (0, 0)),
            pl.BlockSpec((1, 128), lambda i: (0, 0)),
        ],
        out_specs=[blk, col, col],
        out_shape=[
            jax.ShapeDtypeStruct((L, D), jnp.float32),
            jax.ShapeDtypeStruct((L, 1), jnp.float32),
            jax.ShapeDtypeStruct((L, 1), jnp.int32),
        ],
        scratch_shapes=[
            pltpu.VMEM((T_CHUNK, D), jnp.float32),
            pltpu.VMEM((SS, D), jnp.float32),
            pltpu.VMEM((T_CHUNK, D), jnp.float32),
        ],
    )(*xargs, dw, db, w2t, bcb, at, dp, rw_p, rb_p)


def _ffn_body(be_ref, xg_ref, up_ref, ub_ref, dwn_ref, dbn_ref, out_ref, acc_s):
    f = pl.program_id(1)
    n_f = pl.num_programs(1)
    xb = xg_ref[...].astype(jnp.bfloat16)
    hid = jnp.dot(xb, up_ref[0], preferred_element_type=jnp.float32)
    hid = hid + ub_ref[0]
    hid = hid / (1.0 + jnp.exp(-hid))
    part = jnp.dot(hid.astype(jnp.bfloat16), dwn_ref[0],
                   preferred_element_type=jnp.float32)

    @pl.when(f == 0)
    def _():
        acc_s[...] = part + dbn_ref[0]

    @pl.when(f != 0)
    def _():
        acc_s[...] = acc_s[...] + part

    @pl.when(f == n_f - 1)
    def _():
        out_ref[...] = acc_s[...]


def _ffn(be, xg, up_w, ub3, down_w, db3):
    nf = DFF // F_BLK
    grid_spec = pltpu.PrefetchScalarGridSpec(
        num_scalar_prefetch=1,
        grid=(NBLK_MAX, nf),
        in_specs=[
            pl.BlockSpec((M_BLK, D), lambda g, f, be: (g, 0)),
            pl.BlockSpec((1, D, F_BLK), lambda g, f, be: (be[g], 0, f)),
            pl.BlockSpec((1, 1, F_BLK), lambda g, f, be: (be[g] * nf + f, 0, 0)),
            pl.BlockSpec((1, F_BLK, D), lambda g, f, be: (be[g], f, 0)),
            pl.BlockSpec((1, 1, D), lambda g, f, be: (be[g], 0, 0)),
        ],
        out_specs=pl.BlockSpec((M_BLK, D), lambda g, f, be: (g, 0)),
        scratch_shapes=[pltpu.VMEM((M_BLK, D), jnp.float32)],
    )
    return pl.pallas_call(
        _ffn_body,
        grid_spec=grid_spec,
        out_shape=jax.ShapeDtypeStruct((G_PAD, D), jnp.float32),
    )(be, xg, up_w, ub3, down_w, db3)


def _unembed_body(y_ref, moe_ref, tw_ref, g_ref, b_ref, emb_ref, out_ref):
    xb = _ln(y_ref[...] + tw_ref[...] * moe_ref[...], g_ref[...], b_ref[...])
    out_ref[...] = lax.dot_general(
        xb, emb_ref[...], (((1,), (1,)), ((), ())),
        preferred_element_type=jnp.float32)


def _unembed(y, moe, tw, ln_g, ln_b, embed):
    grid = (V // N_BLK, L // M_BLK)
    return pl.pallas_call(
        _unembed_body,
        grid=grid,
        in_specs=[
            pl.BlockSpec((M_BLK, D), lambda n, m: (m, 0)),
            pl.BlockSpec((M_BLK, D), lambda n, m: (m, 0)),
            pl.BlockSpec((M_BLK, 1), lambda n, m: (m, 0)),
            pl.BlockSpec((1, D), lambda n, m: (0, 0)),
            pl.BlockSpec((1, D), lambda n, m: (0, 0)),
            pl.BlockSpec((N_BLK, D), lambda n, m: (n, 0)),
        ],
        out_specs=pl.BlockSpec((M_BLK, N_BLK), lambda n, m: (m, n)),
        out_shape=jax.ShapeDtypeStruct((L, V), jnp.float32),
    )(y, moe, tw, ln_g, ln_b, embed)


def kernel(x, params):
    embed = params['embed']
    idx = x.reshape(-1).astype(jnp.int32)
    h = _sc_gather(embed, idx, L)
    xargs = (h,)
    fuse_ln = False
    for lp in params['layers']:
        at = (-jnp.exp(lp['A_log'])).T
        w2t = jnp.concatenate([lp['B_w'], lp['C_w']], axis=1).T
        bcb = jnp.concatenate([lp['B_b'], lp['C_b']])[:, None]
        rw_p = jnp.pad(lp['router_w'], ((0, 0), (0, 128 - E)))
        rb_p = jnp.pad(lp['router_b'], (0, 128 - E))[None]
        y, tw, ti = _ssm_scan(xargs, lp['delta_w'], lp['delta_b'][None], w2t,
                              bcb, at, lp['Dp'][None], rw_p, rb_p, fuse_ln)
        xg, inv, be = _sc_route_gather(ti.reshape(-1), y)
        ub3 = lp['up_b'].reshape(E * (DFF // F_BLK), 1, F_BLK)
        db3 = lp['down_b'][:, None, :]
        dg = _ffn(be, xg, lp['up_w'].astype(jnp.bfloat16), ub3,
                  lp['down_w'].astype(jnp.bfloat16), db3)
        moe = _sc_gather(dg, inv, L)
        xargs = (y, moe, tw, lp['ln_g'][None], lp['ln_b'][None])
        fuse_ln = True
    logits = _unembed(*xargs, embed)
    return logits[None]


# speedup vs baseline: 1.0770x; 1.0770x over previous
import functools

import jax
import jax.numpy as jnp
from jax import lax
from jax.experimental import pallas as pl
from jax.experimental.pallas import tpu as pltpu
from jax.experimental.pallas import tpu_sc as plsc

V = 32000
D = 768
SS = 16
E = 4
DFF = 4 * D
L = 2048

T_CHUNK = 128
M_BLK = 256
F_BLK = 1024
N_BLK = 3200
G_PAD = L + E * M_BLK
NBLK_MAX = G_PAD // M_BLK


def _sc_gather(table, idx, n_out):
    info = plsc.get_sparse_core_info()
    nw = info.num_cores * info.num_subcores
    b_per_w = n_out // nw
    mesh = plsc.VectorSubcoreMesh(core_axis_name="c", subcore_axis_name="s")

    @functools.partial(
        pl.kernel,
        mesh=mesh,
        out_type=jax.ShapeDtypeStruct((n_out, D), jnp.float32),
        scratch_types=[
            pltpu.VMEM((b_per_w,), jnp.int32),
            pltpu.VMEM((b_per_w, D), jnp.float32),
            pltpu.SemaphoreType.DMA,
        ],
    )
    def k(table_hbm, idx_hbm, out_hbm, idx_v, rows_v, sem):
        wid = lax.axis_index("s") * info.num_cores + lax.axis_index("c")
        base = wid * b_per_w
        pltpu.sync_copy(idx_hbm.at[pl.ds(base, b_per_w)], idx_v)
        pltpu.async_copy(table_hbm.at[idx_v], rows_v, sem).wait()
        pltpu.sync_copy(rows_v, out_hbm.at[pl.ds(base, b_per_w)])

    return k(table, idx)


def _sc_route_gather(ti, y):
    info = plsc.get_sparse_core_info()
    nw = info.num_cores * info.num_subcores
    rows_w = L // nw
    mesh = plsc.VectorSubcoreMesh(core_axis_name="c", subcore_axis_name="s")

    @functools.partial(
        pl.kernel,
        mesh=mesh,
        out_type=[
            jax.ShapeDtypeStruct((G_PAD, D), jnp.float32),
            jax.ShapeDtypeStruct((L,), jnp.int32),
            jax.ShapeDtypeStruct((16,), jnp.int32),
        ],
        scratch_types=[
            pltpu.VMEM((L,), jnp.int32),
            pltpu.VMEM((L,), jnp.int32),
            pltpu.VMEM((16,), jnp.int32),
            pltpu.VMEM((rows_w,), jnp.int32),
            pltpu.VMEM((rows_w, D), jnp.float32),
            pltpu.SemaphoreType.DMA,
        ],
    )
    def k(ti_hbm, y_hbm, xg_hbm, inv_hbm, be_hbm,
          ti_v, inv_v, be_v, idx_v, rows_v, sem):
        wid = lax.axis_index("s") * info.num_cores + lax.axis_index("c")
        pltpu.sync_copy(ti_hbm, ti_v)
        i16 = lax.iota(jnp.int32, 16)
        zv = jnp.zeros((16,), jnp.int32)
        one = jnp.ones((16,), jnp.int32)

        def cnt_body(i, accs):
            eid = ti_v[pl.ds(i * 16, 16)]
            return tuple(accs[e] + jnp.where(eid == e, one, zv)
                         for e in range(E))
        accs = lax.fori_loop(0, L // 16, cnt_body, (zv,) * E)

        def lane_sum(vec):
            s = vec[0]
            for k in range(1, 16):
                s = s + vec[k]
            return s

        cnts = [lane_sum(accs[e]) for e in range(E)]

        nb = [lax.shift_right_logical(c + (M_BLK - 1), 8) for c in cnts]
        cb1 = nb[0]
        cb2 = nb[0] + nb[1]
        cb3 = cb2 + nb[2]
        be_v[...] = (jnp.where(i16 >= cb1, one, zv)
                     + jnp.where(i16 >= cb2, one, zv)
                     + jnp.where(i16 >= cb3, one, zv))

        def sc_body(i, bases):
            b0, b1, b2, b3 = bases
            eid = ti_v[pl.ds(i * 16, 16)]
            eks = [eid[k] for k in range(16)]
            rank = zv
            for k in range(16):
                hit = jnp.logical_and(eid == eks[k], i16 > k)
                rank = rank + jnp.where(hit, one, zv)
            base_vec = jnp.where(eid == 0, b0,
                                 jnp.where(eid == 1, b1,
                                           jnp.where(eid == 2, b2, b3)))
            inv_v[pl.ds(i * 16, 16)] = base_vec + rank
            for k in range(16):
                b0 = b0 + (eks[k] == 0).astype(jnp.int32)
                b1 = b1 + (eks[k] == 1).astype(jnp.int32)
                b2 = b2 + (eks[k] == 2).astype(jnp.int32)
                b3 = b3 + (eks[k] == 3).astype(jnp.int32)
            return (b0, b1, b2, b3)
        lax.fori_loop(0, L // 16, sc_body,
                      (jnp.zeros((), jnp.int32), cb1 * M_BLK, cb2 * M_BLK,
                       cb3 * M_BLK))

        base = wid * rows_w
        def cp(j, c):
            idx_v[pl.ds(j * 16, 16)] = inv_v[pl.ds(base + j * 16, 16)]
            return c
        lax.fori_loop(0, rows_w // 16, cp, 0)
        pltpu.sync_copy(y_hbm.at[pl.ds(base, rows_w)], rows_v)
        pltpu.async_copy(rows_v, xg_hbm.at[idx_v], sem).wait()

        @pl.when(wid == 0)
        def _():
            pltpu.sync_copy(inv_v, inv_hbm)
            pltpu.sync_copy(be_v, be_hbm)

    return k(ti, y)


def _ln(o, g, b):
    mu = jnp.mean(o, axis=1, keepdims=True)
    oc = o - mu
    var = jnp.mean(oc * oc, axis=1, keepdims=True)
    return oc * lax.rsqrt(var + 1e-5) * g + b


def _scan_body(fuse_ln, *refs):
    if fuse_ln:
        (yp_ref, moe_ref, twp_ref, g_ref, b_ref, dw_ref, db_ref, w2t_ref,
         bcb_ref, at_ref, dp_ref, rw_ref, rb_ref,
         y_ref, tw_ref, ti_ref, dstate, h_s, x_s) = refs
    else:
        (x_ref, dw_ref, db_ref, w2t_ref, bcb_ref, at_ref,
         dp_ref, rw_ref, rb_ref, y_ref, tw_ref, ti_ref,
         dstate, h_s, x_s) = refs
    gi = pl.program_id(0)

    @pl.when(gi == 0)
    def _():
        h_s[...] = jnp.zeros_like(h_s)

    if fuse_ln:
        xb = _ln(yp_ref[...] + twp_ref[...] * moe_ref[...],
                 g_ref[...], b_ref[...])
        x_s[...] = xb
        x_ref = x_s
    else:
        xb = x_ref[...]
    delta = jnp.dot(xb, dw_ref[...], preferred_element_type=jnp.float32)
    delta = delta + db_ref[...]
    delta = jnp.log(1.0 + jnp.exp(-jnp.abs(delta))) + jnp.maximum(delta, 0.0)
    dstate[...] = delta
    bct = lax.dot_general(w2t_ref[...], xb, (((1,), (1,)), ((), ())),
                          preferred_element_type=jnp.float32)
    bct = bct + bcb_ref[...]

    at = at_ref[...]
    lane_iota = lax.broadcasted_iota(jnp.int32, (1, T_CHUNK), 1)
    SG = 8

    def group(g, h):
        base = g * SG
        d_g = dstate[pl.ds(base, SG), :]
        x_g = x_ref[pl.ds(base, SG), :]
        ys = []
        for k in range(SG):
            t = base + k
            oh = (lane_iota == t).astype(jnp.float32)
            bc_col = jnp.sum(bct * oh, axis=1, keepdims=True)
            b_col = bc_col[0:SS, :]
            c_col = bc_col[SS:2 * SS, :]
            d_t = d_g[k:k + 1, :]
            x_t = x_g[k:k + 1, :]
            a = jnp.exp(jnp.minimum(d_t * at, 2.0))
            bb = jnp.clip(d_t * b_col, -2.0, 2.0)
            h = a * h + bb * x_t
            h = jnp.clip(h, -100.0, 100.0)
            ys.append(jnp.sum(h * c_col, axis=0, keepdims=True))
        y_ref[pl.ds(base, SG), :] = jnp.concatenate(ys, axis=0)
        return h

    h = lax.fori_loop(0, T_CHUNK // SG, group, h_s[...])
    h_s[...] = h
    yb = y_ref[...] + xb * dp_ref[...]
    y_ref[...] = yb

    logits = jnp.dot(yb, rw_ref[...], preferred_element_type=jnp.float32)
    logits = (logits + rb_ref[...])[:, 0:E]
    mx = jnp.max(logits, axis=1, keepdims=True)
    ex = jnp.exp(logits - mx)
    sm = ex / jnp.sum(ex, axis=1, keepdims=True)
    tw = jnp.max(sm, axis=1, keepdims=True)
    iot = lax.broadcasted_iota(jnp.int32, (T_CHUNK, E), 1)
    ti = jnp.min(jnp.where(sm >= tw, iot, E), axis=1, keepdims=True)
    tw_ref[...] = tw
    ti_ref[...] = ti


def _ssm_scan(xargs, dw, db, w2t, bcb, at, dp, rw_p, rb_p, fuse_ln):
    grid = (L // T_CHUNK,)
    blk = pl.BlockSpec((T_CHUNK, D), lambda i: (i, 0))
    col = pl.BlockSpec((T_CHUNK, 1), lambda i: (i, 0))
    row = pl.BlockSpec((1, D), lambda i: (0, 0))
    if fuse_ln:
        xspecs = [blk, blk, col, row, row]
    else:
        xspecs = [blk]
    return pl.pallas_call(
        functools.partial(_scan_body, fuse_ln),
        grid=grid,
        in_specs=xspecs + [
            pl.BlockSpec((D, D), lambda i: (0, 0)),
            row,
            pl.BlockSpec((2 * SS, D), lambda i: (0, 0)),
            pl.BlockSpec((2 * SS, 1), lambda i: (0, 0)),
            pl.BlockSpec((SS, D), lambda i: (0, 0)),
            row,
            pl.BlockSpec((D, 128), lambda i: (0, 0)),
            pl.BlockSpec((1, 128), lambda i: (0, 0)),
        ],
        out_specs=[blk, col, col],
        out_shape=[
            jax.ShapeDtypeStruct((L, D), jnp.float32),
            jax.ShapeDtypeStruct((L, 1), jnp.float32),
            jax.ShapeDtypeStruct((L, 1), jnp.int32),
        ],
        scratch_shapes=[
            pltpu.VMEM((T_CHUNK, D), jnp.float32),
            pltpu.VMEM((SS, D), jnp.float32),
            pltpu.VMEM((T_CHUNK, D), jnp.float32),
        ],
    )(*xargs, dw, db, w2t, bcb, at, dp, rw_p, rb_p)


def _ffn_body(be_ref, xg_ref, up_ref, ub_ref, dwn_ref, dbn_ref, out_ref, acc_s):
    f = pl.program_id(1)
    n_f = pl.num_programs(1)
    xb = xg_ref[...].astype(jnp.bfloat16)
    hid = jnp.dot(xb, up_ref[0], preferred_element_type=jnp.float32)
    hid = hid + ub_ref[0]
    hid = hid / (1.0 + jnp.exp(-hid))
    part = jnp.dot(hid.astype(jnp.bfloat16), dwn_ref[0],
                   preferred_element_type=jnp.float32)

    @pl.when(f == 0)
    def _():
        acc_s[...] = part + dbn_ref[0]

    @pl.when(f != 0)
    def _():
        acc_s[...] = acc_s[...] + part

    @pl.when(f == n_f - 1)
    def _():
        out_ref[...] = acc_s[...]


def _ffn(be, xg, up_w, ub3, down_w, db3):
    nf = DFF // F_BLK
    grid_spec = pltpu.PrefetchScalarGridSpec(
        num_scalar_prefetch=1,
        grid=(NBLK_MAX, nf),
        in_specs=[
            pl.BlockSpec((M_BLK, D), lambda g, f, be: (g, 0)),
            pl.BlockSpec((1, D, F_BLK), lambda g, f, be: (be[g], 0, f)),
            pl.BlockSpec((1, 1, F_BLK), lambda g, f, be: (be[g] * nf + f, 0, 0)),
            pl.BlockSpec((1, F_BLK, D), lambda g, f, be: (be[g], f, 0)),
            pl.BlockSpec((1, 1, D), lambda g, f, be: (be[g], 0, 0)),
        ],
        out_specs=pl.BlockSpec((M_BLK, D), lambda g, f, be: (g, 0)),
        scratch_shapes=[pltpu.VMEM((M_BLK, D), jnp.float32)],
    )
    return pl.pallas_call(
        _ffn_body,
        grid_spec=grid_spec,
        out_shape=jax.ShapeDtypeStruct((G_PAD, D), jnp.float32),
    )(be, xg, up_w, ub3, down_w, db3)


def _unembed_body(y_ref, moe_ref, tw_ref, g_ref, b_ref, emb_ref, out_ref):
    xb = _ln(y_ref[...] + tw_ref[...] * moe_ref[...], g_ref[...], b_ref[...])
    out_ref[...] = lax.dot_general(
        xb, emb_ref[...], (((1,), (1,)), ((), ())),
        preferred_element_type=jnp.float32)


def _unembed(y, moe, tw, ln_g, ln_b, embed):
    grid = (V // N_BLK, L // M_BLK)
    return pl.pallas_call(
        _unembed_body,
        grid=grid,
        in_specs=[
            pl.BlockSpec((M_BLK, D), lambda n, m: (m, 0)),
            pl.BlockSpec((M_BLK, D), lambda n, m: (m, 0)),
            pl.BlockSpec((M_BLK, 1), lambda n, m: (m, 0)),
            pl.BlockSpec((1, D), lambda n, m: (0, 0)),
            pl.BlockSpec((1, D), lambda n, m: (0, 0)),
            pl.BlockSpec((N_BLK, D), lambda n, m: (n, 0)),
        ],
        out_specs=pl.BlockSpec((M_BLK, N_BLK), lambda n, m: (m, n)),
        out_shape=jax.ShapeDtypeStruct((L, V), jnp.float32),
    )(y, moe, tw, ln_g, ln_b, embed)


def kernel(x, params):
    embed = params['embed']
    idx = x.reshape(-1).astype(jnp.int32)
    h = _sc_gather(embed, idx, L)
    xargs = (h,)
    fuse_ln = False
    for lp in params['layers']:
        at = (-jnp.exp(lp['A_log'])).T
        w2t = jnp.concatenate([lp['B_w'], lp['C_w']], axis=1).T
        bcb = jnp.concatenate([lp['B_b'], lp['C_b']])[:, None]
        rw_p = jnp.pad(lp['router_w'], ((0, 0), (0, 128 - E)))
        rb_p = jnp.pad(lp['router_b'], (0, 128 - E))[None]
        y, tw, ti = _ssm_scan(xargs, lp['delta_w'], lp['delta_b'][None], w2t,
                              bcb, at, lp['Dp'][None], rw_p, rb_p, fuse_ln)
        xg, inv, be = _sc_route_gather(ti.reshape(-1), y)
        ub3 = lp['up_b'].reshape(E * (DFF // F_BLK), 1, F_BLK)
        db3 = lp['down_b'][:, None, :]
        dg = _ffn(be, xg, lp['up_w'].astype(jnp.bfloat16), ub3,
                  lp['down_w'].astype(jnp.bfloat16), db3)
        moe = _sc_gather(dg, inv, L)
        xargs = (y, moe, tw, lp['ln_g'][None], lp['ln_b'][None])
        fuse_ln = True
    logits = _unembed(*xargs, embed)
    return logits[None]


# speedup vs baseline: 1.0996x; 1.0210x over previous
import functools

import jax
import jax.numpy as jnp
from jax import lax
from jax.experimental import pallas as pl
from jax.experimental.pallas import tpu as pltpu
from jax.experimental.pallas import tpu_sc as plsc

V = 32000
D = 768
SS = 16
E = 4
DFF = 4 * D
L = 2048

T_CHUNK = 128
M_BLK = 256
F_BLK = 1024
N_BLK = 3200
G_PAD = L + E * M_BLK
NBLK_MAX = G_PAD // M_BLK


def _sc_gather(table, idx, n_out):
    info = plsc.get_sparse_core_info()
    nw = info.num_cores * info.num_subcores
    b_per_w = n_out // nw
    mesh = plsc.VectorSubcoreMesh(core_axis_name="c", subcore_axis_name="s")

    @functools.partial(
        pl.kernel,
        mesh=mesh,
        out_type=jax.ShapeDtypeStruct((n_out, D), jnp.float32),
        scratch_types=[
            pltpu.VMEM((b_per_w,), jnp.int32),
            pltpu.VMEM((b_per_w, D), jnp.float32),
            pltpu.SemaphoreType.DMA,
        ],
    )
    def k(table_hbm, idx_hbm, out_hbm, idx_v, rows_v, sem):
        wid = lax.axis_index("s") * info.num_cores + lax.axis_index("c")
        base = wid * b_per_w
        pltpu.sync_copy(idx_hbm.at[pl.ds(base, b_per_w)], idx_v)
        pltpu.async_copy(table_hbm.at[idx_v], rows_v, sem).wait()
        pltpu.sync_copy(rows_v, out_hbm.at[pl.ds(base, b_per_w)])

    return k(table, idx)


def _sc_route_gather(ti, y):
    info = plsc.get_sparse_core_info()
    nw = info.num_cores * info.num_subcores
    rows_w = L // nw
    mesh = plsc.VectorSubcoreMesh(core_axis_name="c", subcore_axis_name="s")

    @functools.partial(
        pl.kernel,
        mesh=mesh,
        out_type=[
            jax.ShapeDtypeStruct((G_PAD, D), jnp.float32),
            jax.ShapeDtypeStruct((L,), jnp.int32),
            jax.ShapeDtypeStruct((16,), jnp.int32),
        ],
        scratch_types=[
            pltpu.VMEM((L,), jnp.int32),
            pltpu.VMEM((L,), jnp.int32),
            pltpu.VMEM((16,), jnp.int32),
            pltpu.VMEM((rows_w,), jnp.int32),
            pltpu.VMEM((rows_w, D), jnp.float32),
            pltpu.SemaphoreType.DMA,
        ],
    )
    def k(ti_hbm, y_hbm, xg_hbm, inv_hbm, be_hbm,
          ti_v, inv_v, be_v, idx_v, rows_v, sem):
        wid = lax.axis_index("s") * info.num_cores + lax.axis_index("c")
        pltpu.sync_copy(ti_hbm, ti_v)
        i16 = lax.iota(jnp.int32, 16)
        zv = jnp.zeros((16,), jnp.int32)
        one = jnp.ones((16,), jnp.int32)

        def cnt_body(i, accs):
            eid = ti_v[pl.ds(i * 16, 16)]
            return tuple(accs[e] + jnp.where(eid == e, one, zv)
                         for e in range(E))
        accs = lax.fori_loop(0, L // 16, cnt_body, (zv,) * E)

        def lane_sum(vec):
            s = vec[0]
            for k in range(1, 16):
                s = s + vec[k]
            return s

        cnts = [lane_sum(accs[e]) for e in range(E)]

        nb = [lax.shift_right_logical(c + (M_BLK - 1), 8) for c in cnts]
        cb1 = nb[0]
        cb2 = nb[0] + nb[1]
        cb3 = cb2 + nb[2]
        be_v[...] = (jnp.where(i16 >= cb1, one, zv)
                     + jnp.where(i16 >= cb2, one, zv)
                     + jnp.where(i16 >= cb3, one, zv))

        def sc_body(i, bases):
            b0, b1, b2, b3 = bases
            eid = ti_v[pl.ds(i * 16, 16)]
            eks = [eid[k] for k in range(16)]
            rank = zv
            for k in range(16):
                hit = jnp.logical_and(eid == eks[k], i16 > k)
                rank = rank + jnp.where(hit, one, zv)
            base_vec = jnp.where(eid == 0, b0,
                                 jnp.where(eid == 1, b1,
                                           jnp.where(eid == 2, b2, b3)))
            inv_v[pl.ds(i * 16, 16)] = base_vec + rank
            for k in range(16):
                b0 = b0 + (eks[k] == 0).astype(jnp.int32)
                b1 = b1 + (eks[k] == 1).astype(jnp.int32)
                b2 = b2 + (eks[k] == 2).astype(jnp.int32)
                b3 = b3 + (eks[k] == 3).astype(jnp.int32)
            return (b0, b1, b2, b3)
        lax.fori_loop(0, L // 16, sc_body,
                      (jnp.zeros((), jnp.int32), cb1 * M_BLK, cb2 * M_BLK,
                       cb3 * M_BLK))

        base = wid * rows_w
        def cp(j, c):
            idx_v[pl.ds(j * 16, 16)] = inv_v[pl.ds(base + j * 16, 16)]
            return c
        lax.fori_loop(0, rows_w // 16, cp, 0)
        pltpu.sync_copy(y_hbm.at[pl.ds(base, rows_w)], rows_v)
        pltpu.async_copy(rows_v, xg_hbm.at[idx_v], sem).wait()

        @pl.when(wid == 0)
        def _():
            pltpu.sync_copy(inv_v, inv_hbm)
            pltpu.sync_copy(be_v, be_hbm)

    return k(ti, y)


def _ln(o, g, b):
    mu = jnp.mean(o, axis=1, keepdims=True)
    oc = o - mu
    var = jnp.mean(oc * oc, axis=1, keepdims=True)
    return oc * lax.rsqrt(var + 1e-5) * g + b


def _scan_body(fuse_ln, *refs):
    if fuse_ln:
        (yp_ref, moe_ref, twp_ref, g_ref, b_ref, dw_ref, db_ref, w2t_ref,
         bcb_ref, at_ref, dp_ref, rw_ref, rb_ref,
         y_ref, tw_ref, ti_ref, dstate, h_s, x_s) = refs
    else:
        (x_ref, dw_ref, db_ref, w2t_ref, bcb_ref, at_ref,
         dp_ref, rw_ref, rb_ref, y_ref, tw_ref, ti_ref,
         dstate, h_s, x_s) = refs
    gi = pl.program_id(0)

    @pl.when(gi == 0)
    def _():
        h_s[...] = jnp.zeros_like(h_s)

    if fuse_ln:
        xb = _ln(yp_ref[...] + twp_ref[...] * moe_ref[...],
                 g_ref[...], b_ref[...])
        x_s[...] = xb
        x_ref = x_s
    else:
        xb = x_ref[...]
    delta = jnp.dot(xb, dw_ref[...], preferred_element_type=jnp.float32)
    delta = delta + db_ref[...]
    delta = jnp.log(1.0 + jnp.exp(-jnp.abs(delta))) + jnp.maximum(delta, 0.0)
    dstate[...] = delta
    bct = lax.dot_general(w2t_ref[...], xb, (((1,), (1,)), ((), ())),
                          preferred_element_type=jnp.float32)
    bct = bct + bcb_ref[...]

    at = at_ref[...]
    lane_iota = lax.broadcasted_iota(jnp.int32, (1, T_CHUNK), 1)
    SG = 8

    def group(g, h):
        base = g * SG
        d_g = dstate[pl.ds(base, SG), :]
        x_g = x_ref[pl.ds(base, SG), :]
        ys = []
        for k in range(SG):
            t = base + k
            oh = (lane_iota == t).astype(jnp.float32)
            bc_col = jnp.sum(bct * oh, axis=1, keepdims=True)
            b_col = bc_col[0:SS, :]
            c_col = bc_col[SS:2 * SS, :]
            d_t = d_g[k:k + 1, :]
            x_t = x_g[k:k + 1, :]
            a = jnp.exp(jnp.minimum(d_t * at, 2.0))
            bb = jnp.clip(d_t * b_col, -2.0, 2.0)
            h = a * h + bb * x_t
            h = jnp.clip(h, -100.0, 100.0)
            ys.append(jnp.sum(h * c_col, axis=0, keepdims=True))
        y_ref[pl.ds(base, SG), :] = jnp.concatenate(ys, axis=0)
        return h

    h = lax.fori_loop(0, T_CHUNK // SG, group, h_s[...])
    h_s[...] = h
    yb = y_ref[...] + xb * dp_ref[...]
    y_ref[...] = yb

    logits = jnp.dot(yb, rw_ref[...], preferred_element_type=jnp.float32)
    logits = (logits + rb_ref[...])[:, 0:E]
    mx = jnp.max(logits, axis=1, keepdims=True)
    ex = jnp.exp(logits - mx)
    sm = ex / jnp.sum(ex, axis=1, keepdims=True)
    tw = jnp.max(sm, axis=1, keepdims=True)
    iot = lax.broadcasted_iota(jnp.int32, (T_CHUNK, E), 1)
    ti = jnp.min(jnp.where(sm >= tw, iot, E), axis=1, keepdims=True)
    tw_ref[...] = tw
    ti_ref[...] = ti


def _ssm_scan(xargs, dw, db, w2t, bcb, at, dp, rw_p, rb_p, fuse_ln):
    grid = (L // T_CHUNK,)
    blk = pl.BlockSpec((T_CHUNK, D), lambda i: (i, 0))
    col = pl.BlockSpec((T_CHUNK, 1), lambda i: (i, 0))
    row = pl.BlockSpec((1, D), lambda i: (0, 0))
    if fuse_ln:
        xspecs = [blk, blk, col, row, row]
    else:
        xspecs = [blk]
    return pl.pallas_call(
        functools.partial(_scan_body, fuse_ln),
        grid=grid,
        in_specs=xspecs + [
            pl.BlockSpec((D, D), lambda i: (0, 0)),
            row,
            pl.BlockSpec((2 * SS, D), lambda i: (0, 0)),
            pl.BlockSpec((2 * SS, 1), lambda i: (0, 0)),
            pl.BlockSpec((SS, D), lambda i: (0, 0)),
            row,
            pl.BlockSpec((D, 128), lambda i: (0, 0)),
            pl.BlockSpec((1, 128), lambda i: (0, 0)),
        ],
        out_specs=[blk, col, col],
        out_shape=[
            jax.ShapeDtypeStruct((L, D), jnp.float32),
            jax.ShapeDtypeStruct((L, 1), jnp.float32),
            jax.ShapeDtypeStruct((L, 1), jnp.int32),
        ],
        scratch_shapes=[
            pltpu.VMEM((T_CHUNK, D), jnp.float32),
            pltpu.VMEM((SS, D), jnp.float32),
            pltpu.VMEM((T_CHUNK, D), jnp.float32),
        ],
    )(*xargs, dw, db, w2t, bcb, at, dp, rw_p, rb_p)


def _ffn_body(be_ref, xg_ref, up_ref, ub_ref, dwn_ref, dbn_ref, out_ref, acc_s):
    f = pl.program_id(1)
    n_f = pl.num_programs(1)
    xb = xg_ref[...].astype(jnp.bfloat16)
    hid = jnp.dot(xb, up_ref[0], preferred_element_type=jnp.float32)
    hid = hid + ub_ref[0]
    hid = hid / (1.0 + jnp.exp(-hid))
    part = jnp.dot(hid.astype(jnp.bfloat16), dwn_ref[0],
                   preferred_element_type=jnp.float32)

    @pl.when(f == 0)
    def _():
        acc_s[...] = part + dbn_ref[0]

    @pl.when(f != 0)
    def _():
        acc_s[...] = acc_s[...] + part

    @pl.when(f == n_f - 1)
    def _():
        out_ref[...] = acc_s[...]


def _ffn(be, xg, up_w, ub3, down_w, db3):
    nf = DFF // F_BLK
    grid_spec = pltpu.PrefetchScalarGridSpec(
        num_scalar_prefetch=1,
        grid=(NBLK_MAX, nf),
        in_specs=[
            pl.BlockSpec((M_BLK, D), lambda g, f, be: (g, 0)),
            pl.BlockSpec((1, D, F_BLK), lambda g, f, be: (be[g], 0, f)),
            pl.BlockSpec((1, 1, F_BLK), lambda g, f, be: (be[g] * nf + f, 0, 0)),
            pl.BlockSpec((1, F_BLK, D), lambda g, f, be: (be[g], f, 0)),
            pl.BlockSpec((1, 1, D), lambda g, f, be: (be[g], 0, 0)),
        ],
        out_specs=pl.BlockSpec((M_BLK, D), lambda g, f, be: (g, 0)),
        scratch_shapes=[pltpu.VMEM((M_BLK, D), jnp.float32)],
    )
    return pl.pallas_call(
        _ffn_body,
        grid_spec=grid_spec,
        out_shape=jax.ShapeDtypeStruct((G_PAD, D), jnp.float32),
    )(be, xg, up_w, ub3, down_w, db3)


def _unembed_body(y_ref, moe_ref, tw_ref, g_ref, b_ref, emb_ref, out_ref):
    xb = _ln(y_ref[...] + tw_ref[...] * moe_ref[...], g_ref[...], b_ref[...])
    out_ref[...] = lax.dot_general(
        xb, emb_ref[...], (((1,), (1,)), ((), ())),
        preferred_element_type=jnp.float32)


def _unembed(y, moe, tw, ln_g, ln_b, embed):
    um = 512
    grid = (V // N_BLK, L // um)
    return pl.pallas_call(
        _unembed_body,
        grid=grid,
        in_specs=[
            pl.BlockSpec((um, D), lambda n, m: (m, 0)),
            pl.BlockSpec((um, D), lambda n, m: (m, 0)),
            pl.BlockSpec((um, 1), lambda n, m: (m, 0)),
            pl.BlockSpec((1, D), lambda n, m: (0, 0)),
            pl.BlockSpec((1, D), lambda n, m: (0, 0)),
            pl.BlockSpec((N_BLK, D), lambda n, m: (n, 0)),
        ],
        out_specs=pl.BlockSpec((um, N_BLK), lambda n, m: (m, n)),
        out_shape=jax.ShapeDtypeStruct((L, V), jnp.float32),
    )(y, moe, tw, ln_g, ln_b, embed)


def kernel(x, params):
    embed = params['embed']
    idx = x.reshape(-1).astype(jnp.int32)
    h = _sc_gather(embed, idx, L)
    xargs = (h,)
    fuse_ln = False
    for lp in params['layers']:
        at = (-jnp.exp(lp['A_log'])).T
        w2t = jnp.concatenate([lp['B_w'], lp['C_w']], axis=1).T
        bcb = jnp.concatenate([lp['B_b'], lp['C_b']])[:, None]
        rw_p = jnp.pad(lp['router_w'], ((0, 0), (0, 128 - E)))
        rb_p = jnp.pad(lp['router_b'], (0, 128 - E))[None]
        y, tw, ti = _ssm_scan(xargs, lp['delta_w'], lp['delta_b'][None], w2t,
                              bcb, at, lp['Dp'][None], rw_p, rb_p, fuse_ln)
        xg, inv, be = _sc_route_gather(ti.reshape(-1), y)
        ub3 = lp['up_b'].reshape(E * (DFF // F_BLK), 1, F_BLK)
        db3 = lp['down_b'][:, None, :]
        dg = _ffn(be, xg, lp['up_w'].astype(jnp.bfloat16), ub3,
                  lp['down_w'].astype(jnp.bfloat16), db3)
        moe = _sc_gather(dg, inv, L)
        xargs = (y, moe, tw, lp['ln_g'][None], lp['ln_b'][None])
        fuse_ln = True
    logits = _unembed(*xargs, embed)
    return logits[None]


# speedup vs baseline: 1.1183x; 1.0171x over previous
import functools

import jax
import jax.numpy as jnp
from jax import lax
from jax.experimental import pallas as pl
from jax.experimental.pallas import tpu as pltpu
from jax.experimental.pallas import tpu_sc as plsc

V = 32000
D = 768
SS = 16
E = 4
DFF = 4 * D
L = 2048

T_CHUNK = 256
M_BLK = 256
F_BLK = 1024
N_BLK = 3200
G_PAD = L + E * M_BLK
NBLK_MAX = G_PAD // M_BLK


def _sc_gather(table, idx, n_out):
    info = plsc.get_sparse_core_info()
    nw = info.num_cores * info.num_subcores
    b_per_w = n_out // nw
    mesh = plsc.VectorSubcoreMesh(core_axis_name="c", subcore_axis_name="s")

    @functools.partial(
        pl.kernel,
        mesh=mesh,
        out_type=jax.ShapeDtypeStruct((n_out, D), jnp.float32),
        scratch_types=[
            pltpu.VMEM((b_per_w,), jnp.int32),
            pltpu.VMEM((b_per_w, D), jnp.float32),
            pltpu.SemaphoreType.DMA,
        ],
    )
    def k(table_hbm, idx_hbm, out_hbm, idx_v, rows_v, sem):
        wid = lax.axis_index("s") * info.num_cores + lax.axis_index("c")
        base = wid * b_per_w
        pltpu.sync_copy(idx_hbm.at[pl.ds(base, b_per_w)], idx_v)
        pltpu.async_copy(table_hbm.at[idx_v], rows_v, sem).wait()
        pltpu.sync_copy(rows_v, out_hbm.at[pl.ds(base, b_per_w)])

    return k(table, idx)


def _sc_route_gather(ti, y):
    info = plsc.get_sparse_core_info()
    nw = info.num_cores * info.num_subcores
    rows_w = L // nw
    mesh = plsc.VectorSubcoreMesh(core_axis_name="c", subcore_axis_name="s")

    @functools.partial(
        pl.kernel,
        mesh=mesh,
        out_type=[
            jax.ShapeDtypeStruct((G_PAD, D), jnp.float32),
            jax.ShapeDtypeStruct((L,), jnp.int32),
            jax.ShapeDtypeStruct((16,), jnp.int32),
        ],
        scratch_types=[
            pltpu.VMEM((L,), jnp.int32),
            pltpu.VMEM((L,), jnp.int32),
            pltpu.VMEM((16,), jnp.int32),
            pltpu.VMEM((rows_w,), jnp.int32),
            pltpu.VMEM((rows_w, D), jnp.float32),
            pltpu.SemaphoreType.DMA,
        ],
    )
    def k(ti_hbm, y_hbm, xg_hbm, inv_hbm, be_hbm,
          ti_v, inv_v, be_v, idx_v, rows_v, sem):
        wid = lax.axis_index("s") * info.num_cores + lax.axis_index("c")
        pltpu.sync_copy(ti_hbm, ti_v)
        i16 = lax.iota(jnp.int32, 16)
        zv = jnp.zeros((16,), jnp.int32)
        one = jnp.ones((16,), jnp.int32)

        def cnt_body(i, accs):
            eid = ti_v[pl.ds(i * 16, 16)]
            return tuple(accs[e] + jnp.where(eid == e, one, zv)
                         for e in range(E))
        accs = lax.fori_loop(0, L // 16, cnt_body, (zv,) * E)

        def lane_sum(vec):
            s = vec[0]
            for k in range(1, 16):
                s = s + vec[k]
            return s

        cnts = [lane_sum(accs[e]) for e in range(E)]

        nb = [lax.shift_right_logical(c + (M_BLK - 1), 8) for c in cnts]
        cb1 = nb[0]
        cb2 = nb[0] + nb[1]
        cb3 = cb2 + nb[2]
        be_v[...] = (jnp.where(i16 >= cb1, one, zv)
                     + jnp.where(i16 >= cb2, one, zv)
                     + jnp.where(i16 >= cb3, one, zv))

        def sc_body(i, bases):
            b0, b1, b2, b3 = bases
            eid = ti_v[pl.ds(i * 16, 16)]
            eks = [eid[k] for k in range(16)]
            rank = zv
            for k in range(16):
                hit = jnp.logical_and(eid == eks[k], i16 > k)
                rank = rank + jnp.where(hit, one, zv)
            base_vec = jnp.where(eid == 0, b0,
                                 jnp.where(eid == 1, b1,
                                           jnp.where(eid == 2, b2, b3)))
            inv_v[pl.ds(i * 16, 16)] = base_vec + rank
            for k in range(16):
                b0 = b0 + (eks[k] == 0).astype(jnp.int32)
                b1 = b1 + (eks[k] == 1).astype(jnp.int32)
                b2 = b2 + (eks[k] == 2).astype(jnp.int32)
                b3 = b3 + (eks[k] == 3).astype(jnp.int32)
            return (b0, b1, b2, b3)
        lax.fori_loop(0, L // 16, sc_body,
                      (jnp.zeros((), jnp.int32), cb1 * M_BLK, cb2 * M_BLK,
                       cb3 * M_BLK))

        base = wid * rows_w
        def cp(j, c):
            idx_v[pl.ds(j * 16, 16)] = inv_v[pl.ds(base + j * 16, 16)]
            return c
        lax.fori_loop(0, rows_w // 16, cp, 0)
        pltpu.sync_copy(y_hbm.at[pl.ds(base, rows_w)], rows_v)
        pltpu.async_copy(rows_v, xg_hbm.at[idx_v], sem).wait()

        @pl.when(wid == 0)
        def _():
            pltpu.sync_copy(inv_v, inv_hbm)
            pltpu.sync_copy(be_v, be_hbm)

    return k(ti, y)


def _ln(o, g, b):
    mu = jnp.mean(o, axis=1, keepdims=True)
    oc = o - mu
    var = jnp.mean(oc * oc, axis=1, keepdims=True)
    return oc * lax.rsqrt(var + 1e-5) * g + b


def _scan_body(fuse_ln, *refs):
    if fuse_ln:
        (yp_ref, moe_ref, twp_ref, g_ref, b_ref, dw_ref, db_ref, w2t_ref,
         bcb_ref, at_ref, dp_ref, rw_ref, rb_ref,
         y_ref, tw_ref, ti_ref, dstate, h_s, x_s) = refs
    else:
        (x_ref, dw_ref, db_ref, w2t_ref, bcb_ref, at_ref,
         dp_ref, rw_ref, rb_ref, y_ref, tw_ref, ti_ref,
         dstate, h_s, x_s) = refs
    gi = pl.program_id(0)

    @pl.when(gi == 0)
    def _():
        h_s[...] = jnp.zeros_like(h_s)

    if fuse_ln:
        xb = _ln(yp_ref[...] + twp_ref[...] * moe_ref[...],
                 g_ref[...], b_ref[...])
        x_s[...] = xb
        x_ref = x_s
    else:
        xb = x_ref[...]
    delta = jnp.dot(xb, dw_ref[...], preferred_element_type=jnp.float32)
    delta = delta + db_ref[...]
    delta = jnp.log(1.0 + jnp.exp(-jnp.abs(delta))) + jnp.maximum(delta, 0.0)
    dstate[...] = delta
    bct = lax.dot_general(w2t_ref[...], xb, (((1,), (1,)), ((), ())),
                          preferred_element_type=jnp.float32)
    bct = bct + bcb_ref[...]

    at = at_ref[...]
    lane_iota = lax.broadcasted_iota(jnp.int32, (1, T_CHUNK), 1)
    SG = 8

    def group(g, h):
        base = g * SG
        d_g = dstate[pl.ds(base, SG), :]
        x_g = x_ref[pl.ds(base, SG), :]
        ys = []
        for k in range(SG):
            t = base + k
            oh = (lane_iota == t).astype(jnp.float32)
            bc_col = jnp.sum(bct * oh, axis=1, keepdims=True)
            b_col = bc_col[0:SS, :]
            c_col = bc_col[SS:2 * SS, :]
            d_t = d_g[k:k + 1, :]
            x_t = x_g[k:k + 1, :]
            a = jnp.exp(jnp.minimum(d_t * at, 2.0))
            bb = jnp.clip(d_t * b_col, -2.0, 2.0)
            h = a * h + bb * x_t
            h = jnp.clip(h, -100.0, 100.0)
            ys.append(jnp.sum(h * c_col, axis=0, keepdims=True))
        y_ref[pl.ds(base, SG), :] = jnp.concatenate(ys, axis=0)
        return h

    h = lax.fori_loop(0, T_CHUNK // SG, group, h_s[...])
    h_s[...] = h
    yb = y_ref[...] + xb * dp_ref[...]
    y_ref[...] = yb

    logits = jnp.dot(yb, rw_ref[...], preferred_element_type=jnp.float32)
    logits = (logits + rb_ref[...])[:, 0:E]
    mx = jnp.max(logits, axis=1, keepdims=True)
    ex = jnp.exp(logits - mx)
    sm = ex / jnp.sum(ex, axis=1, keepdims=True)
    tw = jnp.max(sm, axis=1, keepdims=True)
    iot = lax.broadcasted_iota(jnp.int32, (T_CHUNK, E), 1)
    ti = jnp.min(jnp.where(sm >= tw, iot, E), axis=1, keepdims=True)
    tw_ref[...] = tw
    ti_ref[...] = ti


def _ssm_scan(xargs, dw, db, w2t, bcb, at, dp, rw_p, rb_p, fuse_ln):
    grid = (L // T_CHUNK,)
    blk = pl.BlockSpec((T_CHUNK, D), lambda i: (i, 0))
    col = pl.BlockSpec((T_CHUNK, 1), lambda i: (i, 0))
    row = pl.BlockSpec((1, D), lambda i: (0, 0))
    if fuse_ln:
        xspecs = [blk, blk, col, row, row]
    else:
        xspecs = [blk]
    return pl.pallas_call(
        functools.partial(_scan_body, fuse_ln),
        grid=grid,
        in_specs=xspecs + [
            pl.BlockSpec((D, D), lambda i: (0, 0)),
            row,
            pl.BlockSpec((2 * SS, D), lambda i: (0, 0)),
            pl.BlockSpec((2 * SS, 1), lambda i: (0, 0)),
            pl.BlockSpec((SS, D), lambda i: (0, 0)),
            row,
            pl.BlockSpec((D, 128), lambda i: (0, 0)),
            pl.BlockSpec((1, 128), lambda i: (0, 0)),
        ],
        out_specs=[blk, col, col],
        out_shape=[
            jax.ShapeDtypeStruct((L, D), jnp.float32),
            jax.ShapeDtypeStruct((L, 1), jnp.float32),
            jax.ShapeDtypeStruct((L, 1), jnp.int32),
        ],
        scratch_shapes=[
            pltpu.VMEM((T_CHUNK, D), jnp.float32),
            pltpu.VMEM((SS, D), jnp.float32),
            pltpu.VMEM((T_CHUNK, D), jnp.float32),
        ],
    )(*xargs, dw, db, w2t, bcb, at, dp, rw_p, rb_p)


def _ffn_body(be_ref, xg_ref, up_ref, ub_ref, dwn_ref, dbn_ref, out_ref, acc_s):
    f = pl.program_id(1)
    n_f = pl.num_programs(1)
    xb = xg_ref[...].astype(jnp.bfloat16)
    hid = jnp.dot(xb, up_ref[0], preferred_element_type=jnp.float32)
    hid = hid + ub_ref[0]
    hid = hid / (1.0 + jnp.exp(-hid))
    part = jnp.dot(hid.astype(jnp.bfloat16), dwn_ref[0],
                   preferred_element_type=jnp.float32)

    @pl.when(f == 0)
    def _():
        acc_s[...] = part + dbn_ref[0]

    @pl.when(f != 0)
    def _():
        acc_s[...] = acc_s[...] + part

    @pl.when(f == n_f - 1)
    def _():
        out_ref[...] = acc_s[...]


def _ffn(be, xg, up_w, ub3, down_w, db3):
    nf = DFF // F_BLK
    grid_spec = pltpu.PrefetchScalarGridSpec(
        num_scalar_prefetch=1,
        grid=(NBLK_MAX, nf),
        in_specs=[
            pl.BlockSpec((M_BLK, D), lambda g, f, be: (g, 0)),
            pl.BlockSpec((1, D, F_BLK), lambda g, f, be: (be[g], 0, f)),
            pl.BlockSpec((1, 1, F_BLK), lambda g, f, be: (be[g] * nf + f, 0, 0)),
            pl.BlockSpec((1, F_BLK, D), lambda g, f, be: (be[g], f, 0)),
            pl.BlockSpec((1, 1, D), lambda g, f, be: (be[g], 0, 0)),
        ],
        out_specs=pl.BlockSpec((M_BLK, D), lambda g, f, be: (g, 0)),
        scratch_shapes=[pltpu.VMEM((M_BLK, D), jnp.float32)],
    )
    return pl.pallas_call(
        _ffn_body,
        grid_spec=grid_spec,
        out_shape=jax.ShapeDtypeStruct((G_PAD, D), jnp.float32),
    )(be, xg, up_w, ub3, down_w, db3)


def _unembed_body(y_ref, moe_ref, tw_ref, g_ref, b_ref, emb_ref, out_ref):
    xb = _ln(y_ref[...] + tw_ref[...] * moe_ref[...], g_ref[...], b_ref[...])
    out_ref[...] = lax.dot_general(
        xb, emb_ref[...], (((1,), (1,)), ((), ())),
        preferred_element_type=jnp.float32)


def _unembed(y, moe, tw, ln_g, ln_b, embed):
    um = 512
    grid = (V // N_BLK, L // um)
    return pl.pallas_call(
        _unembed_body,
        grid=grid,
        in_specs=[
            pl.BlockSpec((um, D), lambda n, m: (m, 0)),
            pl.BlockSpec((um, D), lambda n, m: (m, 0)),
            pl.BlockSpec((um, 1), lambda n, m: (m, 0)),
            pl.BlockSpec((1, D), lambda n, m: (0, 0)),
            pl.BlockSpec((1, D), lambda n, m: (0, 0)),
            pl.BlockSpec((N_BLK, D), lambda n, m: (n, 0)),
        ],
        out_specs=pl.BlockSpec((um, N_BLK), lambda n, m: (m, n)),
        out_shape=jax.ShapeDtypeStruct((L, V), jnp.float32),
    )(y, moe, tw, ln_g, ln_b, embed)


def kernel(x, params):
    embed = params['embed']
    idx = x.reshape(-1).astype(jnp.int32)
    h = _sc_gather(embed, idx, L)
    xargs = (h,)
    fuse_ln = False
    for lp in params['layers']:
        at = (-jnp.exp(lp['A_log'])).T
        w2t = jnp.concatenate([lp['B_w'], lp['C_w']], axis=1).T
        bcb = jnp.concatenate([lp['B_b'], lp['C_b']])[:, None]
        rw_p = jnp.pad(lp['router_w'], ((0, 0), (0, 128 - E)))
        rb_p = jnp.pad(lp['router_b'], (0, 128 - E))[None]
        y, tw, ti = _ssm_scan(xargs, lp['delta_w'], lp['delta_b'][None], w2t,
                              bcb, at, lp['Dp'][None], rw_p, rb_p, fuse_ln)
        xg, inv, be = _sc_route_gather(ti.reshape(-1), y)
        ub3 = lp['up_b'].reshape(E * (DFF // F_BLK), 1, F_BLK)
        db3 = lp['down_b'][:, None, :]
        dg = _ffn(be, xg, lp['up_w'].astype(jnp.bfloat16), ub3,
                  lp['down_w'].astype(jnp.bfloat16), db3)
        moe = _sc_gather(dg, inv, L)
        xargs = (y, moe, tw, lp['ln_g'][None], lp['ln_b'][None])
        fuse_ln = True
    logits = _unembed(*xargs, embed)
    return logits[None]
